# baseline (speedup 1.0000x reference)
import functools

import jax
import jax.numpy as jnp
from jax import lax
from jax.experimental import pallas as pl
from jax.experimental.pallas import tpu as pltpu
from jax.experimental.pallas import tpu_sc as plsc

_DT = 512


def _sc_gather(spec_labels, table):
    B = spec_labels.shape[0]
    V, D = table.shape
    mesh = plsc.VectorSubcoreMesh(core_axis_name="c", subcore_axis_name="s")

    @functools.partial(
        pl.kernel,
        mesh=mesh,
        out_type=jax.ShapeDtypeStruct((B, D), table.dtype),
        scratch_types=[
            pltpu.VMEM((B,), jnp.int32),
            pltpu.VMEM((B, D), table.dtype),
            pltpu.SemaphoreType.DMA,
        ],
    )
    def gather_kernel(idx_hbm, table_hbm, out_hbm, idx_v, rows_v, sem):
        wid = lax.axis_index("s") * 2 + lax.axis_index("c")

        @pl.when(wid == 0)
        def _():
            pltpu.sync_copy(idx_hbm, idx_v)
            pltpu.async_copy(table_hbm.at[idx_v], rows_v, sem).wait()
            pltpu.sync_copy(rows_v, out_hbm)

    return gather_kernel(spec_labels, table)


def _spec_add_kernel(x_ref, emb_ref, o_ref):
    e = emb_ref[0, 0, 0, :]
    o_ref[...] = x_ref[...] + e[None, :, None]


def kernel(x, spec_labels, table):
    B, D, S = x.shape
    emb = _sc_gather(spec_labels.astype(jnp.int32), table)
    emb3 = emb.reshape(B, D // _DT, 1, _DT)
    return pl.pallas_call(
        _spec_add_kernel,
        grid=(B, D // _DT),
        in_specs=[
            pl.BlockSpec((1, _DT, S), lambda b, d: (b, d, 0)),
            pl.BlockSpec((1, 1, 1, _DT), lambda b, d: (b, d, 0, 0)),
        ],
        out_specs=pl.BlockSpec((1, _DT, S), lambda b, d: (b, d, 0)),
        out_shape=jax.ShapeDtypeStruct((B, D, S), x.dtype),
        compiler_params=pltpu.CompilerParams(
            dimension_semantics=("parallel", "parallel"),
            vmem_limit_bytes=64 * 1024 * 1024,
        ),
    )(x, emb3)


# speedup vs baseline: 1.1452x; 1.1452x over previous
import jax
import jax.numpy as jnp
from jax.experimental import pallas as pl
from jax.experimental.pallas import tpu as pltpu


def _spec_add_kernel(labels_ref, x_ref, emb_ref, o_ref):
    e = emb_ref[0, 0, :]
    o_ref[...] = x_ref[...] + e[None, :, None]


def kernel(x, spec_labels, table):
    B, D, S = x.shape
    Dt = 512
    grid = (B, D // Dt)
    table3 = table.reshape(table.shape[0], 1, D)
    grid_spec = pltpu.PrefetchScalarGridSpec(
        num_scalar_prefetch=1,
        grid=grid,
        in_specs=[
            pl.BlockSpec((1, Dt, S), lambda b, d, labels: (b, d, 0)),
            pl.BlockSpec((1, 1, Dt), lambda b, d, labels: (labels[b], 0, d)),
        ],
        out_specs=pl.BlockSpec((1, Dt, S), lambda b, d, labels: (b, d, 0)),
    )
    return pl.pallas_call(
        _spec_add_kernel,
        grid_spec=grid_spec,
        out_shape=jax.ShapeDtypeStruct((B, D, S), x.dtype),
        compiler_params=pltpu.CompilerParams(
            dimension_semantics=("parallel", "parallel"),
            vmem_limit_bytes=64 * 1024 * 1024,
        ),
    )(spec_labels.astype(jnp.int32), x, table3)
